# baseline (speedup 1.0000x reference)
import jax
import jax.numpy as jnp
from jax import lax
from jax.experimental import pallas as pl
from jax.experimental.pallas import tpu as pltpu
from jax.experimental.pallas import tpu_sc as plsc

N_CLS = 1024
SEQ_LEN = 77
CTX_DIM = 512
N_CTX = 8
SUFFIX = SEQ_LEN - 1 - N_CTX

IDX_W = 80
SOS_N = 8
SUF_N = 72
ROW_SOS = SOS_N - 1
ROW_CTX = SOS_N
ROW_SUF = SOS_N + N_CTX
BUF_H = ROW_SUF + SUF_N

_info = plsc.get_sparse_core_info()
_NC = _info.num_cores
_NS = _info.num_subcores
_NW = _NC * _NS
_CPW = N_CLS // _NW
_NBUF = 2


def _body(idx_hbm, table_hbm, ctx_hbm, out_hbm,
          idx_v, rows0, rows1, gs0, gs1):
    wid = lax.axis_index("s") * _NC + lax.axis_index("c")
    base = wid * _CPW
    rows = (rows0, rows1)
    gsems = (gs0, gs1)

    pltpu.sync_copy(idx_hbm.at[pl.ds(base, _CPW)], idx_v)
    pltpu.sync_copy(ctx_hbm, rows0.at[pl.ds(ROW_CTX, N_CTX)])
    pltpu.sync_copy(ctx_hbm, rows1.at[pl.ds(ROW_CTX, N_CTX)])

    def start_gathers(i, b):
        pltpu.async_copy(table_hbm.at[idx_v.at[i, pl.ds(0, SOS_N)]],
                         rows[b].at[pl.ds(0, SOS_N)], gsems[b])
        pltpu.async_copy(table_hbm.at[idx_v.at[i, pl.ds(SOS_N, SUF_N)]],
                         rows[b].at[pl.ds(ROW_SUF, SUF_N)], gsems[b])

    def wait_gathers(i, b):
        pltpu.make_async_copy(table_hbm.at[idx_v.at[i, pl.ds(0, SOS_N)]],
                              rows[b].at[pl.ds(0, SOS_N)], gsems[b]).wait()
        pltpu.make_async_copy(table_hbm.at[idx_v.at[i, pl.ds(SOS_N, SUF_N)]],
                              rows[b].at[pl.ds(ROW_SUF, SUF_N)],
                              gsems[b]).wait()

    start_gathers(0, 0)

    def step(j, carry):
        for b in range(_NBUF):
            i = j * _NBUF + b

            @pl.when(i + 1 < _CPW)
            def _():
                start_gathers(i + 1, (b + 1) % _NBUF)

            wait_gathers(i, b)
            pltpu.sync_copy(rows[b].at[pl.ds(ROW_SOS, SEQ_LEN)],
                            out_hbm.at[base + i])
        return carry

    lax.fori_loop(0, _CPW // _NBUF, step, 0)


def kernel(tokens, table, ctx):
    idx = jnp.concatenate(
        [jnp.zeros((N_CLS, SOS_N - 1), jnp.int32),
         tokens[:, :1],
         tokens[:, 1 + N_CTX:],
         jnp.zeros((N_CLS, SUF_N - SUFFIX), jnp.int32)], axis=1)
    f = pl.kernel(
        _body,
        out_type=jax.ShapeDtypeStruct((N_CLS, SEQ_LEN, CTX_DIM), jnp.float32),
        mesh=plsc.VectorSubcoreMesh(core_axis_name="c", subcore_axis_name="s"),
        compiler_params=pltpu.CompilerParams(use_tc_tiling_on_sc=False),
        scratch_types=[
            pltpu.VMEM((_CPW, IDX_W), jnp.int32),
            pltpu.VMEM((BUF_H, CTX_DIM), jnp.float32),
            pltpu.VMEM((BUF_H, CTX_DIM), jnp.float32),
            pltpu.SemaphoreType.DMA,
            pltpu.SemaphoreType.DMA,
        ],
    )
    return f(idx, table, ctx)


# speedup vs baseline: 2.3214x; 2.3214x over previous
import jax
import jax.numpy as jnp
from jax import lax
from jax.experimental import pallas as pl
from jax.experimental.pallas import tpu as pltpu
from jax.experimental.pallas import tpu_sc as plsc

N_CLS = 1024
SEQ_LEN = 77
CTX_DIM = 512
N_CTX = 8
SUFFIX = SEQ_LEN - 1 - N_CTX

IDX_W = 80
SOS_N = 8
SUF_N = 72
ROW_SOS = SOS_N - 1
ROW_CTX = SOS_N
ROW_SUF = SOS_N + N_CTX
BUF_H = ROW_SUF + SUF_N

_info = plsc.get_sparse_core_info()
_NC = _info.num_cores
_NS = _info.num_subcores
_NW = _NC * _NS
_CPW = N_CLS // _NW
_NBUF = 2


def _body(idx_hbm, table_hbm, ctx_hbm, out_hbm,
          idx_v, rows0, rows1, gs0, gs1):
    wid = lax.axis_index("s") * _NC + lax.axis_index("c")
    base = wid * _CPW
    rows = (rows0, rows1)
    gsems = (gs0, gs1)

    pltpu.sync_copy(idx_hbm.at[pl.ds(base, _CPW)], idx_v)
    pltpu.sync_copy(ctx_hbm, rows0.at[pl.ds(ROW_CTX, N_CTX)])
    pltpu.sync_copy(ctx_hbm, rows1.at[pl.ds(ROW_CTX, N_CTX)])

    def start_gathers(i, b):
        pltpu.async_copy(table_hbm.at[idx_v.at[i, pl.ds(0, SOS_N)]],
                         rows[b].at[pl.ds(0, SOS_N)], gsems[b])
        pltpu.async_copy(table_hbm.at[idx_v.at[i, pl.ds(SOS_N, SUF_N)]],
                         rows[b].at[pl.ds(ROW_SUF, SUF_N)], gsems[b])

    def wait_gathers(i, b):
        pltpu.make_async_copy(table_hbm.at[idx_v.at[i, pl.ds(0, SOS_N)]],
                              rows[b].at[pl.ds(0, SOS_N)], gsems[b]).wait()
        pltpu.make_async_copy(table_hbm.at[idx_v.at[i, pl.ds(SOS_N, SUF_N)]],
                              rows[b].at[pl.ds(ROW_SUF, SUF_N)],
                              gsems[b]).wait()

    start_gathers(0, 0)

    def step(j, carry):
        for b in range(_NBUF):
            i = j * _NBUF + b

            @pl.when(i + 1 < _CPW)
            def _():
                start_gathers(i + 1, (b + 1) % _NBUF)

            wait_gathers(i, b)
            pltpu.sync_copy(rows[b].at[pl.ds(ROW_SOS, SEQ_LEN)],
                            out_hbm.at[base + i])
        return carry

    lax.fori_loop(0, _CPW // _NBUF, step, 0)


def kernel(tokens, table, ctx):
    idx = jnp.concatenate(
        [tokens[:, 1:SOS_N],
         tokens[:, :1],
         tokens[:, 1 + N_CTX:],
         tokens[:, SEQ_LEN - (SUF_N - SUFFIX):]], axis=1)
    f = pl.kernel(
        _body,
        out_type=jax.ShapeDtypeStruct((N_CLS, SEQ_LEN, CTX_DIM), jnp.float32),
        mesh=plsc.VectorSubcoreMesh(core_axis_name="c", subcore_axis_name="s"),
        compiler_params=pltpu.CompilerParams(use_tc_tiling_on_sc=False),
        scratch_types=[
            pltpu.VMEM((_CPW, IDX_W), jnp.int32),
            pltpu.VMEM((BUF_H, CTX_DIM), jnp.float32),
            pltpu.VMEM((BUF_H, CTX_DIM), jnp.float32),
            pltpu.SemaphoreType.DMA,
            pltpu.SemaphoreType.DMA,
        ],
    )
    return f(idx, table, ctx)


# speedup vs baseline: 4.5102x; 1.9429x over previous
import jax
import jax.numpy as jnp
from jax import lax
from jax.experimental import pallas as pl
from jax.experimental.pallas import tpu as pltpu
from jax.experimental.pallas import tpu_sc as plsc

N_CLS = 1024
SEQ_LEN = 77
CTX_DIM = 512
N_CTX = 8
LANES = 16
NCH = CTX_DIM // LANES

B_N = 64
C_N = 5
MAIN = 72

_info = plsc.get_sparse_core_info()
_NC = _info.num_cores
_NS = _info.num_subcores
_NW = _NC * _NS
_CPW = N_CLS // _NW
_NBUF = 2


def _copy_row(dst_ref, dst_r, src_ref, src_r):
    for k in range(NCH):
        dst_ref[dst_r, pl.ds(k * LANES, LANES)] = (
            src_ref[src_r, pl.ds(k * LANES, LANES)])


def _body(idxa_hbm, idxb_hbm, idxc_hbm, table_hbm, ctx_hbm, out_hbm,
          idxa_v, idxb_v, idxc_v, ctx_v,
          rows0, rows1, sa0, sa1, sc0, sc1, gs0, gs1):
    wid = lax.axis_index("s") * _NC + lax.axis_index("c")
    base = wid * _CPW
    rows = (rows0, rows1)
    sas = (sa0, sa1)
    scs = (sc0, sc1)
    gsems = (gs0, gs1)

    pltpu.sync_copy(idxa_hbm.at[pl.ds(base, _CPW)], idxa_v)
    pltpu.sync_copy(idxb_hbm.at[pl.ds(base, _CPW)], idxb_v)
    pltpu.sync_copy(idxc_hbm.at[pl.ds(base, _CPW)], idxc_v)
    pltpu.sync_copy(ctx_hbm, ctx_v)
    for b in range(_NBUF):
        for r in range(N_CTX - 1):
            _copy_row(rows[b], 1 + r, ctx_v, r)

    def start_gathers(i, b):
        pltpu.async_copy(table_hbm.at[idxa_v.at[i]], sas[b], gsems[b])
        pltpu.async_copy(table_hbm.at[idxb_v.at[i]],
                         rows[b].at[pl.ds(N_CTX, B_N)], gsems[b])
        pltpu.async_copy(table_hbm.at[idxc_v.at[i]], scs[b], gsems[b])

    def wait_gathers(i, b):
        pltpu.make_async_copy(table_hbm.at[idxa_v.at[i]], sas[b],
                              gsems[b]).wait()
        pltpu.make_async_copy(table_hbm.at[idxb_v.at[i]],
                              rows[b].at[pl.ds(N_CTX, B_N)],
                              gsems[b]).wait()
        pltpu.make_async_copy(table_hbm.at[idxc_v.at[i]], scs[b],
                              gsems[b]).wait()

    start_gathers(0, 0)

    def step(j, carry):
        for b in range(_NBUF):
            i = j * _NBUF + b

            @pl.when(i + 1 < _CPW)
            def _():
                start_gathers(i + 1, (b + 1) % _NBUF)

            wait_gathers(i, b)
            _copy_row(rows[b], 0, sas[b], 0)
            _copy_row(rows[b], N_CTX, ctx_v, N_CTX - 1)
            c = base + i
            pltpu.sync_copy(rows[b], out_hbm.at[c, pl.ds(0, MAIN)])
            pltpu.sync_copy(scs[b], out_hbm.at[c, pl.ds(MAIN, C_N)])
        return carry

    lax.fori_loop(0, _CPW // _NBUF, step, 0)


def kernel(tokens, table, ctx):
    idxa = tokens[:, :1]
    idxb = jnp.concatenate(
        [tokens[:, SEQ_LEN - 1:], tokens[:, 1 + N_CTX:MAIN]], axis=1)
    idxc = tokens[:, MAIN:]
    f = pl.kernel(
        _body,
        out_type=jax.ShapeDtypeStruct((N_CLS, SEQ_LEN, CTX_DIM), jnp.float32),
        mesh=plsc.VectorSubcoreMesh(core_axis_name="c", subcore_axis_name="s"),
        scratch_types=[
            pltpu.VMEM((_CPW, 1), jnp.int32),
            pltpu.VMEM((_CPW, B_N), jnp.int32),
            pltpu.VMEM((_CPW, C_N), jnp.int32),
            pltpu.VMEM((N_CTX, CTX_DIM), jnp.float32),
            pltpu.VMEM((MAIN, CTX_DIM), jnp.float32),
            pltpu.VMEM((MAIN, CTX_DIM), jnp.float32),
            pltpu.VMEM((1, CTX_DIM), jnp.float32),
            pltpu.VMEM((1, CTX_DIM), jnp.float32),
            pltpu.VMEM((C_N, CTX_DIM), jnp.float32),
            pltpu.VMEM((C_N, CTX_DIM), jnp.float32),
            pltpu.SemaphoreType.DMA,
            pltpu.SemaphoreType.DMA,
        ],
    )
    return f(idxa, idxb, idxc, table, ctx)
